# baseline (speedup 1.0000x reference)
import functools

import jax
import jax.numpy as jnp
from jax import lax
from jax.experimental import pallas as pl
from jax.experimental.pallas import tpu as pltpu
from jax.experimental.pallas import tpu_sc as plsc

_FEATURES = 32
_NC = 2
_NS = 16
_NW = _NC * _NS
_CH = 128
_SUPER = 5
_ROWS_PER_SUPER = _CH * _SUPER


def _retile_table(tt):
    n_feat, n_vocab = tt.shape
    bn = 16384
    grid = (n_vocab + bn - 1) // bn

    def body(x_ref, o_ref):
        eye = jnp.eye(n_feat, dtype=jnp.float32)
        t = lax.dot_general(
            x_ref[...], eye, (((0,), (0,)), ((), ())),
            precision=lax.Precision.HIGHEST,
            preferred_element_type=jnp.float32).reshape(bn // 4, 4, 32)
        o_ref[...] = jnp.concatenate([t[:, q, :] for q in range(4)], axis=1)

    return pl.pallas_call(
        body,
        grid=(grid,),
        in_specs=[pl.BlockSpec((n_feat, bn), lambda g: (0, g))],
        out_specs=pl.BlockSpec((bn // 4, 128), lambda g: (g, 0)),
        out_shape=jax.ShapeDtypeStruct((n_vocab * n_feat // 128, 128),
                                       jnp.float32),
    )(tt)


def _embed_lookup(n_flat):
    n_per_w = n_flat // _NW
    n_chunks = n_per_w // _CH
    n_super = n_chunks // _SUPER
    mesh = plsc.VectorSubcoreMesh(core_axis_name="c", subcore_axis_name="s")

    @functools.partial(
        pl.kernel,
        out_type=jax.ShapeDtypeStruct((n_flat, _FEATURES), jnp.float32),
        mesh=mesh,
        scratch_types=[
            pltpu.VMEM((n_chunks, _CH), jnp.int32),
            pltpu.VMEM((_ROWS_PER_SUPER, _FEATURES), jnp.float32),
            pltpu.SemaphoreType.DMA,
        ],
        compiler_params=pltpu.CompilerParams(use_tc_tiling_on_sc=False),
    )
    def body(idx_hbm, table_hbm, out_hbm, idx_v, rows_v, sem):
        wid = lax.axis_index("s") * _NC + lax.axis_index("c")
        base = wid * n_per_w
        pltpu.sync_copy(idx_hbm.at[wid], idx_v)

        def step(s, carry):
            copies = []
            for k in range(_SUPER):
                c = s * _SUPER + k
                copies.append(pltpu.async_copy(
                    table_hbm.at[idx_v.at[c]],
                    rows_v.at[pl.ds(k * _CH, _CH)],
                    sem))
            for cp in copies:
                cp.wait()
            pltpu.sync_copy(
                rows_v,
                out_hbm.at[pl.ds(base + s * _ROWS_PER_SUPER, _ROWS_PER_SUPER)])
            return carry

        lax.fori_loop(0, n_super, step, 0)

    return body


def kernel(inputs, embedding):
    b, s = inputs.shape
    n_flat = b * s
    idx3 = inputs.reshape(_NW, n_flat // _NW // _CH, _CH)
    table_rm = _retile_table(jnp.swapaxes(embedding, 0, 1))
    table = table_rm.reshape(embedding.shape)
    out = _embed_lookup(n_flat)(idx3, table)
    return out.reshape(b, s, _FEATURES)


# speedup vs baseline: 1.3758x; 1.3758x over previous
import functools

import jax
import jax.numpy as jnp
from jax import lax
from jax.experimental import pallas as pl
from jax.experimental.pallas import tpu as pltpu
from jax.experimental.pallas import tpu_sc as plsc

_FEATURES = 32
_NC = 2
_NS = 16
_NW = _NC * _NS
_CH = 128
_SUPER = 5
_ROWS_PER_SUPER = _CH * _SUPER


def _retile_table(tt):
    n_feat, n_vocab = tt.shape
    bn = 16384
    grid = (n_vocab + bn - 1) // bn

    def body(x_ref, o_ref):
        eye = jnp.eye(n_feat, dtype=jnp.float32)
        t = lax.dot_general(
            x_ref[...], eye, (((0,), (0,)), ((), ())),
            precision=lax.Precision.DEFAULT,
            preferred_element_type=jnp.float32).reshape(bn // 4, 4, 32)
        o_ref[...] = jnp.concatenate([t[:, q, :] for q in range(4)], axis=1)

    return pl.pallas_call(
        body,
        grid=(grid,),
        in_specs=[pl.BlockSpec((n_feat, bn), lambda g: (0, g))],
        out_specs=pl.BlockSpec((bn // 4, 128), lambda g: (g, 0)),
        out_shape=jax.ShapeDtypeStruct((n_vocab * n_feat // 128, 128),
                                       jnp.float32),
    )(tt)


def _embed_lookup(n_flat):
    n_per_w = n_flat // _NW
    n_chunks = n_per_w // _CH
    n_super = n_chunks // _SUPER
    mesh = plsc.VectorSubcoreMesh(core_axis_name="c", subcore_axis_name="s")

    @functools.partial(
        pl.kernel,
        out_type=jax.ShapeDtypeStruct((n_flat, _FEATURES), jnp.float32),
        mesh=mesh,
        scratch_types=[
            pltpu.VMEM((n_chunks, _CH), jnp.int32),
            pltpu.VMEM((_ROWS_PER_SUPER, _FEATURES), jnp.float32),
            pltpu.SemaphoreType.DMA,
        ],
        compiler_params=pltpu.CompilerParams(use_tc_tiling_on_sc=False),
    )
    def body(idx_hbm, table_hbm, out_hbm, idx_v, rows_v, sem):
        wid = lax.axis_index("s") * _NC + lax.axis_index("c")
        base = wid * n_per_w
        pltpu.sync_copy(idx_hbm.at[wid], idx_v)

        def step(s, carry):
            copies = []
            for k in range(_SUPER):
                c = s * _SUPER + k
                copies.append(pltpu.async_copy(
                    table_hbm.at[idx_v.at[c]],
                    rows_v.at[pl.ds(k * _CH, _CH)],
                    sem))
            for cp in copies:
                cp.wait()
            pltpu.sync_copy(
                rows_v,
                out_hbm.at[pl.ds(base + s * _ROWS_PER_SUPER, _ROWS_PER_SUPER)])
            return carry

        lax.fori_loop(0, n_super, step, 0)

    return body


def kernel(inputs, embedding):
    b, s = inputs.shape
    n_flat = b * s
    idx3 = inputs.reshape(_NW, n_flat // _NW // _CH, _CH)
    table_rm = _retile_table(jnp.swapaxes(embedding, 0, 1))
    table = table_rm.reshape(embedding.shape)
    out = _embed_lookup(n_flat)(idx3, table)
    return out.reshape(b, s, _FEATURES)


# speedup vs baseline: 1.3970x; 1.0154x over previous
import functools

import jax
import jax.numpy as jnp
from jax import lax
from jax.experimental import pallas as pl
from jax.experimental.pallas import tpu as pltpu
from jax.experimental.pallas import tpu_sc as plsc

_FEATURES = 32
_NC = 2
_NS = 16
_NW = _NC * _NS
_CH = 128
_SUPER = 5
_ROWS_PER_SUPER = _CH * _SUPER


def _retile_table(tt):
    n_feat, n_vocab = tt.shape
    bn = 16384
    grid = (n_vocab + bn - 1) // bn

    def body(x_ref, o_ref):
        t = jnp.transpose(x_ref[...], (1, 0)).reshape(bn // 4, 4, 32)
        o_ref[...] = jnp.concatenate([t[:, q, :] for q in range(4)], axis=1)

    return pl.pallas_call(
        body,
        grid=(grid,),
        in_specs=[pl.BlockSpec((n_feat, bn), lambda g: (0, g))],
        out_specs=pl.BlockSpec((bn // 4, 128), lambda g: (g, 0)),
        out_shape=jax.ShapeDtypeStruct((n_vocab * n_feat // 128, 128),
                                       jnp.float32),
    )(tt)


def _embed_lookup(n_flat):
    n_per_w = n_flat // _NW
    n_chunks = n_per_w // _CH
    n_super = n_chunks // _SUPER
    mesh = plsc.VectorSubcoreMesh(core_axis_name="c", subcore_axis_name="s")

    @functools.partial(
        pl.kernel,
        out_type=jax.ShapeDtypeStruct((n_flat, _FEATURES), jnp.float32),
        mesh=mesh,
        scratch_types=[
            pltpu.VMEM((n_chunks, _CH), jnp.int32),
            pltpu.VMEM((_ROWS_PER_SUPER, _FEATURES), jnp.float32),
            pltpu.SemaphoreType.DMA,
        ],
        compiler_params=pltpu.CompilerParams(use_tc_tiling_on_sc=False),
    )
    def body(idx_hbm, table_hbm, out_hbm, idx_v, rows_v, sem):
        wid = lax.axis_index("s") * _NC + lax.axis_index("c")
        base = wid * n_per_w
        pltpu.sync_copy(idx_hbm.at[wid], idx_v)

        def step(s, carry):
            copies = []
            for k in range(_SUPER):
                c = s * _SUPER + k
                copies.append(pltpu.async_copy(
                    table_hbm.at[idx_v.at[c]],
                    rows_v.at[pl.ds(k * _CH, _CH)],
                    sem))
            for cp in copies:
                cp.wait()
            pltpu.sync_copy(
                rows_v,
                out_hbm.at[pl.ds(base + s * _ROWS_PER_SUPER, _ROWS_PER_SUPER)])
            return carry

        lax.fori_loop(0, n_super, step, 0)

    return body


def kernel(inputs, embedding):
    b, s = inputs.shape
    n_flat = b * s
    idx3 = inputs.reshape(_NW, n_flat // _NW // _CH, _CH)
    table_rm = _retile_table(jnp.swapaxes(embedding, 0, 1))
    table = table_rm.reshape(embedding.shape)
    out = _embed_lookup(n_flat)(idx3, table)
    return out.reshape(b, s, _FEATURES)


# speedup vs baseline: 1.8347x; 1.3134x over previous
import functools

import jax
import jax.numpy as jnp
from jax import lax
from jax.experimental import pallas as pl
from jax.experimental.pallas import tpu as pltpu
from jax.experimental.pallas import tpu_sc as plsc

_FEATURES = 32
_NC = 2
_NS = 16
_NW = _NC * _NS
_CH = 128
_SUPER = 5
_ROWS_PER_SUPER = _CH * _SUPER


def _retile_table(tt):
    n_feat, n_vocab = tt.shape
    bn = 16384
    grid = (n_vocab + bn - 1) // bn

    def body(x_ref, o_ref):
        t = jnp.transpose(x_ref[...], (1, 0)).reshape(bn // 4, 4, 32)
        o_ref[...] = jnp.concatenate([t[:, q, :] for q in range(4)], axis=1)

    return pl.pallas_call(
        body,
        grid=(grid,),
        in_specs=[pl.BlockSpec((n_feat, bn), lambda g: (0, g))],
        out_specs=pl.BlockSpec((bn // 4, 128), lambda g: (g, 0)),
        out_shape=jax.ShapeDtypeStruct((n_vocab * n_feat // 128, 128),
                                       jnp.float32),
    )(tt)


def _embed_lookup(n_step, n_blk):
    mesh = plsc.VectorSubcoreMesh(core_axis_name="c", subcore_axis_name="s")

    @functools.partial(
        pl.kernel,
        out_type=jax.ShapeDtypeStruct((n_step, _NW, _CH, _FEATURES),
                                      jnp.float32),
        mesh=mesh,
        scratch_types=[
            pltpu.VMEM((n_step, _CH), jnp.int32),
            pltpu.VMEM((_SUPER, _CH, _FEATURES), jnp.float32),
            pltpu.SemaphoreType.DMA,
        ],
        compiler_params=pltpu.CompilerParams(use_tc_tiling_on_sc=False),
    )
    def body(idx_hbm, table_hbm, out_hbm, idx_v, rows_v, sem):
        wid = lax.axis_index("s") * _NC + lax.axis_index("c")
        pltpu.sync_copy(idx_hbm.at[:, pl.ds(wid * _CH, _CH)], idx_v)

        def step(t, carry):
            copies = []
            for k in range(_SUPER):
                s = t * _SUPER + k
                copies.append(pltpu.async_copy(
                    table_hbm.at[idx_v.at[s]], rows_v.at[k], sem))
            for cp in copies:
                cp.wait()
            for k in range(_SUPER):
                s = t * _SUPER + k
                pltpu.sync_copy(rows_v.at[k], out_hbm.at[s, wid])
            return carry

        lax.fori_loop(0, n_step // _SUPER, step, 0)

    return body


def kernel(inputs, embedding):
    b, s = inputs.shape
    idxT = jnp.swapaxes(inputs, 0, 1)
    table_rm = _retile_table(jnp.swapaxes(embedding, 0, 1))
    table = table_rm.reshape(embedding.shape)
    out4 = _embed_lookup(s, b // _CH)(idxT, table)
    return out4.transpose(1, 2, 0, 3).reshape(b, s, _FEATURES)


# speedup vs baseline: 1.8456x; 1.0059x over previous
import functools

import jax
import jax.numpy as jnp
from jax import lax
from jax.experimental import pallas as pl
from jax.experimental.pallas import tpu as pltpu
from jax.experimental.pallas import tpu_sc as plsc

_FEATURES = 32
_NC = 2
_NS = 16
_NW = _NC * _NS
_CH = 128
_SUPER = 5
_ROWS_PER_SUPER = _CH * _SUPER


def _retile_table(tt):
    n_feat, n_vocab = tt.shape
    bn = 32768
    grid = (n_vocab + bn - 1) // bn

    def body(x_ref, o_ref):
        t = jnp.transpose(x_ref[...], (1, 0)).reshape(bn // 4, 4, 32)
        o_ref[...] = jnp.concatenate([t[:, q, :] for q in range(4)], axis=1)

    return pl.pallas_call(
        body,
        grid=(grid,),
        in_specs=[pl.BlockSpec((n_feat, bn), lambda g: (0, g))],
        out_specs=pl.BlockSpec((bn // 4, 128), lambda g: (g, 0)),
        out_shape=jax.ShapeDtypeStruct((n_vocab * n_feat // 128, 128),
                                       jnp.float32),
    )(tt)


def _embed_lookup(n_step, n_blk):
    mesh = plsc.VectorSubcoreMesh(core_axis_name="c", subcore_axis_name="s")

    @functools.partial(
        pl.kernel,
        out_type=jax.ShapeDtypeStruct((n_step, _NW, _CH, _FEATURES),
                                      jnp.float32),
        mesh=mesh,
        scratch_types=[
            pltpu.VMEM((n_step, _CH), jnp.int32),
            pltpu.VMEM((_SUPER, _CH, _FEATURES), jnp.float32),
            pltpu.SemaphoreType.DMA,
        ],
        compiler_params=pltpu.CompilerParams(use_tc_tiling_on_sc=False),
    )
    def body(idx_hbm, table_hbm, out_hbm, idx_v, rows_v, sem):
        wid = lax.axis_index("s") * _NC + lax.axis_index("c")
        pltpu.sync_copy(idx_hbm.at[:, pl.ds(wid * _CH, _CH)], idx_v)

        def step(t, carry):
            copies = []
            for k in range(_SUPER):
                s = t * _SUPER + k
                copies.append(pltpu.async_copy(
                    table_hbm.at[idx_v.at[s]], rows_v.at[k], sem))
            for cp in copies:
                cp.wait()
            for k in range(_SUPER):
                s = t * _SUPER + k
                pltpu.sync_copy(rows_v.at[k], out_hbm.at[s, wid])
            return carry

        lax.fori_loop(0, n_step // _SUPER, step, 0)

    return body


def kernel(inputs, embedding):
    b, s = inputs.shape
    idxT = jnp.swapaxes(inputs, 0, 1)
    table_rm = _retile_table(jnp.swapaxes(embedding, 0, 1))
    table = table_rm.reshape(embedding.shape)
    out4 = _embed_lookup(s, b // _CH)(idxT, table)
    return out4.transpose(1, 2, 0, 3).reshape(b, s, _FEATURES)
